# baseline (speedup 1.0000x reference)
import functools

import jax
import jax.numpy as jnp
from jax import lax
from jax.experimental import pallas as pl
from jax.experimental.pallas import tpu as pltpu
from jax.experimental.pallas import tpu_sc as plsc

N = 10000
D = 128
E = 320000
NC = 2
NS = 16
NW = NC * NS
K = 128
EPT = ((E + NW * K - 1) // (NW * K)) * K
EPAD = NW * EPT
CHUNKS = EPT // K
RPT = ((N + NW) // NS + 7) // 8 * 8
NPAD = NS * RPT
C0 = 69
C1 = 2 * CHUNKS - C0
CMAX = max(C0, C1)


def _seg_pipe(feat, dstp, acc_sp, src_all, dst_b, msg_b, sem_g, sem_s,
              base_e, chunks):
    pltpu.sync_copy(dstp.at[pl.ds(base_e, K)], dst_b[0])
    gathers = [None, None]
    scatters = [None, None]
    gathers[0] = pltpu.async_copy(
        feat.at[src_all.at[pl.ds(0, K)]], msg_b[0], sem_g)
    for i in range(chunks):
        b = i % 2
        nb = 1 - b
        gathers[b].wait()
        if scatters[nb] is not None:
            scatters[nb].wait()
        if i + 1 < chunks:
            e1 = base_e + (i + 1) * K
            pltpu.sync_copy(dstp.at[pl.ds(e1, K)], dst_b[nb])
            gathers[nb] = pltpu.async_copy(
                feat.at[src_all.at[pl.ds((i + 1) * K, K)]], msg_b[nb], sem_g)
        scatters[b] = pltpu.async_copy(
            msg_b[b], acc_sp.at[dst_b[b]], sem_s, add=True)
    scatters[(chunks - 1) % 2].wait()


def _seg_body(feat, srcp, dstp, z_acc,
              acc_out,
              acc_sp, src_all, dst_v0, dst_v1, msg_v0, msg_v1, sem_g, sem_s):
    c = lax.axis_index("c")
    s = lax.axis_index("s")
    row0 = s * RPT
    msg_b = (msg_v0, msg_v1)
    dst_b = (dst_v0, dst_v1)
    base0 = s * (C0 * K)
    base1 = NS * (C0 * K) + s * (C1 * K)
    @pl.when(c == 0)
    def _():
        pltpu.sync_copy(srcp.at[pl.ds(base0, C0 * K)],
                        src_all.at[pl.ds(0, C0 * K)])
    @pl.when(c == 1)
    def _():
        pltpu.sync_copy(srcp.at[pl.ds(base1, C1 * K)],
                        src_all.at[pl.ds(0, C1 * K)])
    pltpu.sync_copy(z_acc, acc_sp.at[pl.ds(row0, RPT)])
    plsc.subcore_barrier()
    @pl.when(c == 0)
    def _():
        _seg_pipe(feat, dstp, acc_sp, src_all, dst_b, msg_b, sem_g, sem_s,
                  base0, C0)
    @pl.when(c == 1)
    def _():
        _seg_pipe(feat, dstp, acc_sp, src_all, dst_b, msg_b, sem_g, sem_s,
                  base1, C1)
    plsc.subcore_barrier()
    pltpu.sync_copy(acc_sp.at[pl.ds(row0, RPT)],
                    acc_out.at[c, pl.ds(row0, RPT)])


def _cnt_body(dstp, z_acc, ones_hbm,
              cnt_out,
              cnt_sp, dst_all, ones_v, sem_s):
    c = lax.axis_index("c")
    s = lax.axis_index("s")
    row0 = s * RPT
    base_e = (c * NS + s) * EPT
    pltpu.sync_copy(ones_hbm, ones_v)
    pltpu.sync_copy(dstp.at[pl.ds(base_e, EPT)], dst_all)
    pltpu.sync_copy(z_acc, cnt_sp.at[pl.ds(row0, RPT)])
    plsc.subcore_barrier()
    scatters = []
    for i in range(CHUNKS):
        scatters.append(pltpu.async_copy(
            ones_v, cnt_sp.at[dst_all.at[pl.ds(i * K, K)]], sem_s, add=True))
    for d in scatters:
        d.wait()
    plsc.subcore_barrier()
    pltpu.sync_copy(cnt_sp.at[pl.ds(row0, RPT)],
                    cnt_out.at[c, pl.ds(row0, RPT)])


@functools.lru_cache(maxsize=1)
def _sc_kernels():
    mesh = plsc.VectorSubcoreMesh(
        core_axis_name="c", subcore_axis_name="s",
        num_cores=NC, num_subcores=NS)
    seg = pl.kernel(
        _seg_body,
        out_type=jax.ShapeDtypeStruct((NC, NPAD, D), jnp.float32),
        mesh=mesh,
        scratch_types=[
            pltpu.VMEM_SHARED((NPAD, D), jnp.float32),
            pltpu.VMEM((CMAX * K,), jnp.int32),
            pltpu.VMEM((K,), jnp.int32),
            pltpu.VMEM((K,), jnp.int32),
            pltpu.VMEM((K, D), jnp.float32),
            pltpu.VMEM((K, D), jnp.float32),
            pltpu.SemaphoreType.DMA,
            pltpu.SemaphoreType.DMA,
        ],
    )
    cnt = pl.kernel(
        _cnt_body,
        out_type=jax.ShapeDtypeStruct((NC, NPAD, D), jnp.float32),
        mesh=mesh,
        scratch_types=[
            pltpu.VMEM_SHARED((NPAD, D), jnp.float32),
            pltpu.VMEM((EPT,), jnp.int32),
            pltpu.VMEM((K, D), jnp.float32),
            pltpu.SemaphoreType.DMA,
        ],
    )
    return seg, cnt


def _tc1_body(x_ref, s0a, s0b, ca, cb, wskT, bsk, wl0T, bl0, wr0T,
              h1_ref, xs_ref):
    x = x_ref[...]
    colcnt = jnp.sum((x == 1.0).astype(jnp.float32), axis=0)
    keep = (colcnt != 1.0).astype(jnp.float32)
    wsk = wskT[...] * keep[:, None]
    xs = jnp.maximum(
        jnp.dot(x, wsk, preferred_element_type=jnp.float32) + bsk[...], 0.0)
    cnt = jnp.maximum(ca[...] + cb[...], 1.0)
    mean0 = (s0a[...] + s0b[...]) / cnt
    h1 = jnp.maximum(
        jnp.dot(mean0, wl0T[...], preferred_element_type=jnp.float32)
        + bl0[...]
        + jnp.dot(x, wr0T[...], preferred_element_type=jnp.float32), 0.0)
    h1_ref[...] = h1
    xs_ref[...] = xs


def _tc2_body(h1_ref, xs_ref, s1a, s1b, ca, cb, wl1T, bl1, wr1T,
              w1T, b1r, w2T, b2r, out_ref):
    h1 = h1_ref[...]
    cnt = jnp.maximum(ca[...] + cb[...], 1.0)
    mean1 = (s1a[...] + s1b[...]) / cnt
    h2 = (jnp.dot(mean1, wl1T[...], preferred_element_type=jnp.float32)
          + bl1[...]
          + jnp.dot(h1, wr1T[...], preferred_element_type=jnp.float32))
    h = xs_ref[...] + h1 + h2
    h = jnp.maximum(
        jnp.dot(h, w1T[...], preferred_element_type=jnp.float32) + b1r[...],
        0.0)
    z = jnp.dot(h, w2T[...], preferred_element_type=jnp.float32) + b2r[...]
    out_ref[...] = 1.0 / (1.0 + jnp.exp(-z))


_tc1 = pl.pallas_call(
    _tc1_body,
    out_shape=(jax.ShapeDtypeStruct((N, D), jnp.float32),
               jax.ShapeDtypeStruct((N, D), jnp.float32)),
)

_tc2 = pl.pallas_call(
    _tc2_body,
    out_shape=jax.ShapeDtypeStruct((N, 1), jnp.float32),
)


def kernel(x, edge_index, W_skip, b_skip, Wl0, bl0, Wr0, Wl1, bl1, Wr1,
           W1, b1, W2, b2):
    src = edge_index[0]
    dst = edge_index[1]
    pad = EPAD - E
    srcp = jnp.concatenate([src, jnp.zeros((pad,), jnp.int32)])
    dstp = jnp.concatenate([dst, jnp.full((pad,), N, jnp.int32)])
    z_acc = jnp.zeros((RPT, D), jnp.float32)
    ones_hbm = jnp.ones((K, D), jnp.float32)

    sc_seg, sc_cnt = _sc_kernels()
    cntp = sc_cnt(dstp, z_acc, ones_hbm)
    s0p = sc_seg(x, srcp, dstp, z_acc)

    ca, cb = cntp[0, :N, 0:1], cntp[1, :N, 0:1]
    h1, xs = _tc1(x, s0p[0, :N], s0p[1, :N], ca, cb,
                  W_skip.T, b_skip.reshape(1, D),
                  Wl0.T, bl0.reshape(1, D), Wr0.T)

    s1p = sc_seg(h1, srcp, dstp, z_acc)

    out = _tc2(h1, xs, s1p[0, :N], s1p[1, :N], ca, cb,
               Wl1.T, bl1.reshape(1, D), Wr1.T,
               W1.T, b1.reshape(1, D), W2.T, b2.reshape(1, 1))
    return out


# speedup vs baseline: 1.0890x; 1.0890x over previous
import functools

import jax
import jax.numpy as jnp
from jax import lax
from jax.experimental import pallas as pl
from jax.experimental.pallas import tpu as pltpu
from jax.experimental.pallas import tpu_sc as plsc

N = 10000
D = 128
E = 320000
NC = 2
NS = 16
NW = NC * NS
K = 128
EPT = ((E + NW * K - 1) // (NW * K)) * K
EPAD = NW * EPT
CHUNKS = EPT // K
RPT = ((N + NW) // NS + 7) // 8 * 8
NPAD = NS * RPT
C0 = 89
C1 = 2 * CHUNKS - C0
CMAX = max(C0, C1)


def _seg_pipe(feat, dstp, acc_sp, src_all, dst_b, msg_b, sem_g, sem_s,
              base_e, chunks):
    pltpu.sync_copy(dstp.at[pl.ds(base_e, K)], dst_b[0])
    gathers = [None, None]
    scatters = [None, None]
    gathers[0] = pltpu.async_copy(
        feat.at[src_all.at[pl.ds(0, K)]], msg_b[0], sem_g)
    for i in range(chunks):
        b = i % 2
        nb = 1 - b
        gathers[b].wait()
        if scatters[nb] is not None:
            scatters[nb].wait()
        if i + 1 < chunks:
            e1 = base_e + (i + 1) * K
            pltpu.sync_copy(dstp.at[pl.ds(e1, K)], dst_b[nb])
            gathers[nb] = pltpu.async_copy(
                feat.at[src_all.at[pl.ds((i + 1) * K, K)]], msg_b[nb], sem_g)
        scatters[b] = pltpu.async_copy(
            msg_b[b], acc_sp.at[dst_b[b]], sem_s, add=True)
    scatters[(chunks - 1) % 2].wait()


def _seg_body(feat, srcp, dstp, z_acc,
              acc_out,
              acc_sp, src_all, dst_v0, dst_v1, msg_v0, msg_v1, sem_g, sem_s):
    c = lax.axis_index("c")
    s = lax.axis_index("s")
    row0 = s * RPT
    msg_b = (msg_v0, msg_v1)
    dst_b = (dst_v0, dst_v1)
    base0 = s * (C0 * K)
    base1 = NS * (C0 * K) + s * (C1 * K)
    @pl.when(c == 0)
    def _():
        pltpu.sync_copy(srcp.at[pl.ds(base0, C0 * K)],
                        src_all.at[pl.ds(0, C0 * K)])
    @pl.when(c == 1)
    def _():
        pltpu.sync_copy(srcp.at[pl.ds(base1, C1 * K)],
                        src_all.at[pl.ds(0, C1 * K)])
    pltpu.sync_copy(z_acc, acc_sp.at[pl.ds(row0, RPT)])
    plsc.subcore_barrier()
    @pl.when(c == 0)
    def _():
        _seg_pipe(feat, dstp, acc_sp, src_all, dst_b, msg_b, sem_g, sem_s,
                  base0, C0)
    @pl.when(c == 1)
    def _():
        _seg_pipe(feat, dstp, acc_sp, src_all, dst_b, msg_b, sem_g, sem_s,
                  base1, C1)
    plsc.subcore_barrier()
    pltpu.sync_copy(acc_sp.at[pl.ds(row0, RPT)],
                    acc_out.at[c, pl.ds(row0, RPT)])


def _cnt_body(dstp, z_acc, ones_hbm,
              cnt_out,
              cnt_sp, dst_all, ones_v, sem_s):
    c = lax.axis_index("c")
    s = lax.axis_index("s")
    row0 = s * RPT
    base_e = (c * NS + s) * EPT
    pltpu.sync_copy(ones_hbm, ones_v)
    pltpu.sync_copy(dstp.at[pl.ds(base_e, EPT)], dst_all)
    pltpu.sync_copy(z_acc, cnt_sp.at[pl.ds(row0, RPT)])
    plsc.subcore_barrier()
    scatters = []
    for i in range(CHUNKS):
        scatters.append(pltpu.async_copy(
            ones_v, cnt_sp.at[dst_all.at[pl.ds(i * K, K)]], sem_s, add=True))
    for d in scatters:
        d.wait()
    plsc.subcore_barrier()
    pltpu.sync_copy(cnt_sp.at[pl.ds(row0, RPT)],
                    cnt_out.at[c, pl.ds(row0, RPT)])


@functools.lru_cache(maxsize=1)
def _sc_kernels():
    mesh = plsc.VectorSubcoreMesh(
        core_axis_name="c", subcore_axis_name="s",
        num_cores=NC, num_subcores=NS)
    seg = pl.kernel(
        _seg_body,
        out_type=jax.ShapeDtypeStruct((NC, NPAD, D), jnp.float32),
        mesh=mesh,
        scratch_types=[
            pltpu.VMEM_SHARED((NPAD, D), jnp.float32),
            pltpu.VMEM((CMAX * K,), jnp.int32),
            pltpu.VMEM((K,), jnp.int32),
            pltpu.VMEM((K,), jnp.int32),
            pltpu.VMEM((K, D), jnp.float32),
            pltpu.VMEM((K, D), jnp.float32),
            pltpu.SemaphoreType.DMA,
            pltpu.SemaphoreType.DMA,
        ],
    )
    cnt = pl.kernel(
        _cnt_body,
        out_type=jax.ShapeDtypeStruct((NC, NPAD, D), jnp.float32),
        mesh=mesh,
        scratch_types=[
            pltpu.VMEM_SHARED((NPAD, D), jnp.float32),
            pltpu.VMEM((EPT,), jnp.int32),
            pltpu.VMEM((K, D), jnp.float32),
            pltpu.SemaphoreType.DMA,
        ],
    )
    return seg, cnt


def _tc1_body(x_ref, s0a, s0b, ca, cb, wskT, bsk, wl0T, bl0, wr0T,
              h1_ref, xs_ref):
    x = x_ref[...]
    colcnt = jnp.sum((x == 1.0).astype(jnp.float32), axis=0)
    keep = (colcnt != 1.0).astype(jnp.float32)
    wsk = wskT[...] * keep[:, None]
    xs = jnp.maximum(
        jnp.dot(x, wsk, preferred_element_type=jnp.float32) + bsk[...], 0.0)
    cnt = jnp.maximum(ca[...] + cb[...], 1.0)
    mean0 = (s0a[...] + s0b[...]) / cnt
    h1 = jnp.maximum(
        jnp.dot(mean0, wl0T[...], preferred_element_type=jnp.float32)
        + bl0[...]
        + jnp.dot(x, wr0T[...], preferred_element_type=jnp.float32), 0.0)
    h1_ref[...] = h1
    xs_ref[...] = xs


def _tc2_body(h1_ref, xs_ref, s1a, s1b, ca, cb, wl1T, bl1, wr1T,
              w1T, b1r, w2T, b2r, out_ref):
    h1 = h1_ref[...]
    cnt = jnp.maximum(ca[...] + cb[...], 1.0)
    mean1 = (s1a[...] + s1b[...]) / cnt
    h2 = (jnp.dot(mean1, wl1T[...], preferred_element_type=jnp.float32)
          + bl1[...]
          + jnp.dot(h1, wr1T[...], preferred_element_type=jnp.float32))
    h = xs_ref[...] + h1 + h2
    h = jnp.maximum(
        jnp.dot(h, w1T[...], preferred_element_type=jnp.float32) + b1r[...],
        0.0)
    z = jnp.dot(h, w2T[...], preferred_element_type=jnp.float32) + b2r[...]
    out_ref[...] = 1.0 / (1.0 + jnp.exp(-z))


_tc1 = pl.pallas_call(
    _tc1_body,
    out_shape=(jax.ShapeDtypeStruct((N, D), jnp.float32),
               jax.ShapeDtypeStruct((N, D), jnp.float32)),
)

_tc2 = pl.pallas_call(
    _tc2_body,
    out_shape=jax.ShapeDtypeStruct((N, 1), jnp.float32),
)


def kernel(x, edge_index, W_skip, b_skip, Wl0, bl0, Wr0, Wl1, bl1, Wr1,
           W1, b1, W2, b2):
    src = edge_index[0]
    dst = edge_index[1]
    pad = EPAD - E
    srcp = jnp.concatenate([src, jnp.zeros((pad,), jnp.int32)])
    dstp = jnp.concatenate([dst, jnp.full((pad,), N, jnp.int32)])
    z_acc = jnp.zeros((RPT, D), jnp.float32)
    ones_hbm = jnp.ones((K, D), jnp.float32)

    sc_seg, sc_cnt = _sc_kernels()
    cntp = sc_cnt(dstp, z_acc, ones_hbm)
    s0p = sc_seg(x, srcp, dstp, z_acc)

    ca, cb = cntp[0, :N, 0:1], cntp[1, :N, 0:1]
    h1, xs = _tc1(x, s0p[0, :N], s0p[1, :N], ca, cb,
                  W_skip.T, b_skip.reshape(1, D),
                  Wl0.T, bl0.reshape(1, D), Wr0.T)

    s1p = sc_seg(h1, srcp, dstp, z_acc)

    out = _tc2(h1, xs, s1p[0, :N], s1p[1, :N], ca, cb,
               Wl1.T, bl1.reshape(1, D), Wr1.T,
               W1.T, b1.reshape(1, D), W2.T, b2.reshape(1, 1))
    return out


# speedup vs baseline: 1.1488x; 1.0549x over previous
import functools

import jax
import jax.numpy as jnp
from jax import lax
from jax.experimental import pallas as pl
from jax.experimental.pallas import tpu as pltpu
from jax.experimental.pallas import tpu_sc as plsc

N = 10000
D = 128
E = 320000
NC = 2
NS = 16
NW = NC * NS
K = 128
EPT = ((E + NW * K - 1) // (NW * K)) * K
EPAD = NW * EPT
CHUNKS = EPT // K
RPT = ((N + NW) // NS + 7) // 8 * 8
NPAD = NS * RPT
C0 = 100
C1 = 2 * CHUNKS - C0
CMAX = max(C0, C1)


def _seg_pipe(feat, dstp, acc_sp, src_all, dst_b, msg_b, sem_g, sem_s,
              base_e, chunks):
    pltpu.sync_copy(dstp.at[pl.ds(base_e, K)], dst_b[0])
    gathers = [None, None]
    scatters = [None, None]
    gathers[0] = pltpu.async_copy(
        feat.at[src_all.at[pl.ds(0, K)]], msg_b[0], sem_g)
    for i in range(chunks):
        b = i % 2
        nb = 1 - b
        gathers[b].wait()
        if scatters[nb] is not None:
            scatters[nb].wait()
        if i + 1 < chunks:
            e1 = base_e + (i + 1) * K
            pltpu.sync_copy(dstp.at[pl.ds(e1, K)], dst_b[nb])
            gathers[nb] = pltpu.async_copy(
                feat.at[src_all.at[pl.ds((i + 1) * K, K)]], msg_b[nb], sem_g)
        scatters[b] = pltpu.async_copy(
            msg_b[b], acc_sp.at[dst_b[b]], sem_s, add=True)
    scatters[(chunks - 1) % 2].wait()


def _seg_body(feat, srcp, dstp, z_acc,
              acc_out,
              acc_sp, src_all, dst_v0, dst_v1, msg_v0, msg_v1, sem_g, sem_s):
    c = lax.axis_index("c")
    s = lax.axis_index("s")
    row0 = s * RPT
    msg_b = (msg_v0, msg_v1)
    dst_b = (dst_v0, dst_v1)
    base0 = s * (C0 * K)
    base1 = NS * (C0 * K) + s * (C1 * K)
    @pl.when(c == 0)
    def _():
        pltpu.sync_copy(srcp.at[pl.ds(base0, C0 * K)],
                        src_all.at[pl.ds(0, C0 * K)])
    @pl.when(c == 1)
    def _():
        pltpu.sync_copy(srcp.at[pl.ds(base1, C1 * K)],
                        src_all.at[pl.ds(0, C1 * K)])
    pltpu.sync_copy(z_acc, acc_sp.at[pl.ds(row0, RPT)])
    plsc.subcore_barrier()
    @pl.when(c == 0)
    def _():
        _seg_pipe(feat, dstp, acc_sp, src_all, dst_b, msg_b, sem_g, sem_s,
                  base0, C0)
    @pl.when(c == 1)
    def _():
        _seg_pipe(feat, dstp, acc_sp, src_all, dst_b, msg_b, sem_g, sem_s,
                  base1, C1)
    plsc.subcore_barrier()
    pltpu.sync_copy(acc_sp.at[pl.ds(row0, RPT)],
                    acc_out.at[c, pl.ds(row0, RPT)])


def _cnt_body(dstp, z_acc, ones_hbm,
              cnt_out,
              cnt_sp, dst_all, ones_v, sem_s):
    c = lax.axis_index("c")
    s = lax.axis_index("s")
    row0 = s * RPT
    base_e = (c * NS + s) * EPT
    pltpu.sync_copy(ones_hbm, ones_v)
    pltpu.sync_copy(dstp.at[pl.ds(base_e, EPT)], dst_all)
    pltpu.sync_copy(z_acc, cnt_sp.at[pl.ds(row0, RPT)])
    plsc.subcore_barrier()
    scatters = []
    for i in range(CHUNKS):
        scatters.append(pltpu.async_copy(
            ones_v, cnt_sp.at[dst_all.at[pl.ds(i * K, K)]], sem_s, add=True))
    for d in scatters:
        d.wait()
    plsc.subcore_barrier()
    pltpu.sync_copy(cnt_sp.at[pl.ds(row0, RPT)],
                    cnt_out.at[c, pl.ds(row0, RPT)])


@functools.lru_cache(maxsize=1)
def _sc_kernels():
    mesh = plsc.VectorSubcoreMesh(
        core_axis_name="c", subcore_axis_name="s",
        num_cores=NC, num_subcores=NS)
    seg = pl.kernel(
        _seg_body,
        out_type=jax.ShapeDtypeStruct((NC, NPAD, D), jnp.float32),
        mesh=mesh,
        scratch_types=[
            pltpu.VMEM_SHARED((NPAD, D), jnp.float32),
            pltpu.VMEM((CMAX * K,), jnp.int32),
            pltpu.VMEM((K,), jnp.int32),
            pltpu.VMEM((K,), jnp.int32),
            pltpu.VMEM((K, D), jnp.float32),
            pltpu.VMEM((K, D), jnp.float32),
            pltpu.SemaphoreType.DMA,
            pltpu.SemaphoreType.DMA,
        ],
    )
    cnt = pl.kernel(
        _cnt_body,
        out_type=jax.ShapeDtypeStruct((NC, NPAD, D), jnp.float32),
        mesh=mesh,
        scratch_types=[
            pltpu.VMEM_SHARED((NPAD, D), jnp.float32),
            pltpu.VMEM((EPT,), jnp.int32),
            pltpu.VMEM((K, D), jnp.float32),
            pltpu.SemaphoreType.DMA,
        ],
    )
    return seg, cnt


def _tc1_body(x_ref, s0a, s0b, ca, cb, wskT, bsk, wl0T, bl0, wr0T,
              h1_ref, xs_ref):
    x = x_ref[...]
    colcnt = jnp.sum((x == 1.0).astype(jnp.float32), axis=0)
    keep = (colcnt != 1.0).astype(jnp.float32)
    wsk = wskT[...] * keep[:, None]
    xs = jnp.maximum(
        jnp.dot(x, wsk, preferred_element_type=jnp.float32) + bsk[...], 0.0)
    cnt = jnp.maximum(ca[...] + cb[...], 1.0)
    mean0 = (s0a[...] + s0b[...]) / cnt
    h1 = jnp.maximum(
        jnp.dot(mean0, wl0T[...], preferred_element_type=jnp.float32)
        + bl0[...]
        + jnp.dot(x, wr0T[...], preferred_element_type=jnp.float32), 0.0)
    h1_ref[...] = h1
    xs_ref[...] = xs


def _tc2_body(h1_ref, xs_ref, s1a, s1b, ca, cb, wl1T, bl1, wr1T,
              w1T, b1r, w2T, b2r, out_ref):
    h1 = h1_ref[...]
    cnt = jnp.maximum(ca[...] + cb[...], 1.0)
    mean1 = (s1a[...] + s1b[...]) / cnt
    h2 = (jnp.dot(mean1, wl1T[...], preferred_element_type=jnp.float32)
          + bl1[...]
          + jnp.dot(h1, wr1T[...], preferred_element_type=jnp.float32))
    h = xs_ref[...] + h1 + h2
    h = jnp.maximum(
        jnp.dot(h, w1T[...], preferred_element_type=jnp.float32) + b1r[...],
        0.0)
    z = jnp.dot(h, w2T[...], preferred_element_type=jnp.float32) + b2r[...]
    out_ref[...] = 1.0 / (1.0 + jnp.exp(-z))


_tc1 = pl.pallas_call(
    _tc1_body,
    out_shape=(jax.ShapeDtypeStruct((N, D), jnp.float32),
               jax.ShapeDtypeStruct((N, D), jnp.float32)),
)

_tc2 = pl.pallas_call(
    _tc2_body,
    out_shape=jax.ShapeDtypeStruct((N, 1), jnp.float32),
)


def kernel(x, edge_index, W_skip, b_skip, Wl0, bl0, Wr0, Wl1, bl1, Wr1,
           W1, b1, W2, b2):
    src = edge_index[0]
    dst = edge_index[1]
    pad = EPAD - E
    srcp = jnp.concatenate([src, jnp.zeros((pad,), jnp.int32)])
    dstp = jnp.concatenate([dst, jnp.full((pad,), N, jnp.int32)])
    z_acc = jnp.zeros((RPT, D), jnp.float32)
    ones_hbm = jnp.ones((K, D), jnp.float32)

    sc_seg, sc_cnt = _sc_kernels()
    cntp = sc_cnt(dstp, z_acc, ones_hbm)
    s0p = sc_seg(x, srcp, dstp, z_acc)

    ca, cb = cntp[0, :N, 0:1], cntp[1, :N, 0:1]
    h1, xs = _tc1(x, s0p[0, :N], s0p[1, :N], ca, cb,
                  W_skip.T, b_skip.reshape(1, D),
                  Wl0.T, bl0.reshape(1, D), Wr0.T)

    s1p = sc_seg(h1, srcp, dstp, z_acc)

    out = _tc2(h1, xs, s1p[0, :N], s1p[1, :N], ca, cb,
               Wl1.T, bl1.reshape(1, D), Wr1.T,
               W1.T, b1.reshape(1, D), W2.T, b2.reshape(1, 1))
    return out
